# Initial kernel scaffold; baseline (speedup 1.0000x reference)
import jax
import jax.numpy as jnp
from jax.experimental import pallas as pl


def kernel(x_list, edge_index, W_lin, b_lin, W_gcn, W_out, b_out):
    raise NotImplementedError("write your pallas kernel here")


# speedup vs baseline: 7.1023x; 7.1023x over previous
import functools
import math

import jax
import jax.numpy as jnp
from jax import lax
from jax.experimental import pallas as pl
from jax.experimental.pallas import tpu as pltpu
from jax.experimental.pallas import tpu_sc as plsc

_NC = 2
_NS = 16
_CHUNK = 128
_ALPHA = 0.1
_THETA = 0.5


def _cdiv(a, b):
    return (a + b - 1) // b


def _pre_body(n, br, hw, x_ref, w_ref, b_ref, o_ref):
    i = pl.program_id(0)
    h = jnp.dot(x_ref[...], w_ref[...], preferred_element_type=jnp.float32)
    h = jnp.maximum(h + b_ref[...], 0.0)
    rows = i * br + lax.broadcasted_iota(jnp.int32, h.shape, 0)
    h = jnp.where(rows < n, h, 0.0)
    o_ref[0] = h[:, :hw]
    o_ref[1] = h[:, hw:]


def _layer_body(beta, hw, agg_ref, h0_ref, w_ref, o_ref):
    a = jnp.concatenate([agg_ref[0], agg_ref[1]], axis=1)
    h0 = jnp.concatenate([h0_ref[0], h0_ref[1]], axis=1)
    s = (1.0 - _ALPHA) * a + _ALPHA * h0
    sw = jnp.dot(s, w_ref[...], preferred_element_type=jnp.float32)
    h = jnp.maximum((1.0 - beta) * s + beta * sw, 0.0)
    o_ref[0] = h[:, :hw]
    o_ref[1] = h[:, hw:]


def _last_body(beta, agg_ref, h0_ref, w_ref, wo_ref, bo_ref, o_ref):
    a = jnp.concatenate([agg_ref[0], agg_ref[1]], axis=1)
    h0 = jnp.concatenate([h0_ref[0], h0_ref[1]], axis=1)
    s = (1.0 - _ALPHA) * a + _ALPHA * h0
    sw = jnp.dot(s, w_ref[...], preferred_element_type=jnp.float32)
    h = jnp.maximum((1.0 - beta) * s + beta * sw, 0.0)
    o_ref[...] = jnp.dot(h, wo_ref[...], preferred_element_type=jnp.float32) + bo_ref[...]


def _make_scatter(n_pad, hw, chunks):
    rpt = n_pad // _NS
    mesh = plsc.VectorSubcoreMesh(core_axis_name="c", subcore_axis_name="s",
                                  num_cores=_NC, num_subcores=_NS)

    def body(h_hbm, src_hbm, dst_hbm, z_hbm, out_hbm, src_v, dst_v, rows_v,
             acc_sh, sem):
        c = lax.axis_index("c")
        s = lax.axis_index("s")
        pltpu.sync_copy(z_hbm, acc_sh.at[pl.ds(s * rpt, rpt)])
        pltpu.sync_copy(src_hbm.at[c, s], src_v)
        pltpu.sync_copy(dst_hbm.at[s], dst_v)
        plsc.subcore_barrier()

        def step(j, carry):
            pltpu.async_copy(h_hbm.at[src_v.at[j]], rows_v, sem).wait()
            pltpu.sync_copy(rows_v, acc_sh.at[dst_v.at[j]], add=True)
            return carry

        lax.fori_loop(0, chunks, step, 0)
        plsc.subcore_barrier()
        pltpu.sync_copy(acc_sh.at[pl.ds(s * rpt, rpt)],
                        out_hbm.at[c, pl.ds(s * rpt, rpt)])

    return pl.kernel(
        body,
        out_type=jax.ShapeDtypeStruct((_NC, n_pad, hw), jnp.float32),
        mesh=mesh,
        scratch_types=[
            pltpu.VMEM((chunks, _CHUNK), jnp.int32),
            pltpu.VMEM((chunks, _CHUNK), jnp.int32),
            pltpu.VMEM((_CHUNK, hw), jnp.float32),
            pltpu.VMEM_SHARED((n_pad, hw), jnp.float32),
            pltpu.SemaphoreType.DMA,
        ],
        compiler_params=pltpu.CompilerParams(use_tc_tiling_on_sc=False),
    )


def kernel(x_list, edge_index, W_lin, b_lin, W_gcn, W_out, b_out):
    k_, n, d_in = x_list.shape
    hw = W_lin.shape[2]
    kh = k_ * hw
    l_ = W_gcn.shape[0]
    c_out = W_out.shape[1]
    e = edge_index.shape[1]

    n_pad = _cdiv(n, 2048) * 2048
    e_pad = _cdiv(e, _NS * _CHUNK * 2) * _NS * _CHUNK * 2
    chunks = e_pad // (_NS * _CHUNK)

    x2 = jnp.transpose(x_list, (1, 0, 2)).reshape(n, k_ * d_in)
    x2 = jnp.pad(x2, ((0, n_pad - n), (0, 0)))
    w_blk = jnp.zeros((k_ * d_in, kh), jnp.float32)
    for k in range(k_):
        w_blk = w_blk.at[k * d_in:(k + 1) * d_in, k * hw:(k + 1) * hw].set(W_lin[k])
    b_blk = b_lin.reshape(1, kh)

    pad_e = e_pad - e
    fill = jnp.full((pad_e,), n, jnp.int32)
    src1 = jnp.concatenate([edge_index[0], fill])
    src2 = jnp.stack([src1, src1 + n_pad]).reshape(_NC, _NS, chunks, _CHUNK)
    dst = jnp.concatenate([edge_index[1], fill]).reshape(_NS, chunks, _CHUNK)
    zeros_h = jnp.zeros((n_pad // _NS, hw), jnp.float32)

    br = 1024
    grid = (n_pad // br,)
    tab_spec = pl.BlockSpec((_NC, br, hw), lambda i: (0, i, 0))
    tab_shape = jax.ShapeDtypeStruct((_NC, n_pad, hw), jnp.float32)

    h0_tab = pl.pallas_call(
        functools.partial(_pre_body, n, br, hw),
        grid=grid,
        in_specs=[pl.BlockSpec((br, k_ * d_in), lambda i: (i, 0)),
                  pl.BlockSpec((k_ * d_in, kh), lambda i: (0, 0)),
                  pl.BlockSpec((1, kh), lambda i: (0, 0))],
        out_specs=tab_spec,
        out_shape=tab_shape,
    )(x2, w_blk, b_blk)

    scatter = _make_scatter(n_pad, hw, chunks)

    wo_pad = jnp.zeros((kh, 128), jnp.float32).at[:, :c_out].set(W_out)
    bo_pad = jnp.zeros((1, 128), jnp.float32).at[0, :c_out].set(b_out)

    h_tab = h0_tab
    out_pad = None
    for l in range(l_):
        beta = math.log(_THETA / (l + 1) + 1.0)
        agg = scatter(h_tab.reshape(_NC * n_pad, hw), src2, dst, zeros_h)
        if l < l_ - 1:
            h_tab = pl.pallas_call(
                functools.partial(_layer_body, beta, hw),
                grid=grid,
                in_specs=[tab_spec, tab_spec,
                          pl.BlockSpec((kh, kh), lambda i: (0, 0))],
                out_specs=tab_spec,
                out_shape=tab_shape,
            )(agg, h0_tab, W_gcn[l])
        else:
            out_pad = pl.pallas_call(
                functools.partial(_last_body, beta),
                grid=grid,
                in_specs=[tab_spec, tab_spec,
                          pl.BlockSpec((kh, kh), lambda i: (0, 0)),
                          pl.BlockSpec((kh, 128), lambda i: (0, 0)),
                          pl.BlockSpec((1, 128), lambda i: (0, 0))],
                out_specs=pl.BlockSpec((br, 128), lambda i: (i, 0)),
                out_shape=jax.ShapeDtypeStruct((n_pad, 128), jnp.float32),
            )(agg, h0_tab, W_gcn[l], wo_pad, bo_pad)

    return out_pad[:n, :c_out]
